# baseline (speedup 1.0000x reference)
import functools

import jax
import jax.numpy as jnp
from jax import lax
from jax.experimental import pallas as pl
from jax.experimental.pallas import tpu as pltpu
from jax.experimental.pallas import tpu_sc as plsc

N_NODES = 10000
N_PAD = 10240
N_EDGES = 320000
E_PAD = 327680
D = 128
G = 64
DH = 64
NSUB = 16
EPT = E_PAD // NSUB
NCHUNK = EPT // 128
ROWS_PER_TILE = N_PAD // NSUB

_f32 = jnp.float32
_i32 = jnp.int32


_R = 1024


def _dense_tail(xv, w_ref, avs_ref, avd_ref,
                h_ref, as_ref, ad_ref, ms_ref, md_ref):
    i = pl.program_id(0)
    h = jnp.dot(xv, w_ref[...], preferred_element_type=_f32)
    h_ref[0] = h[:, :DH]
    h_ref[1] = h[:, DH:]
    asv = jnp.sum(h * avs_ref[...], axis=1, keepdims=True)
    adv = jnp.sum(h * avd_ref[...], axis=1, keepdims=True)
    as_ref[...] = asv
    ad_ref[...] = adv
    bs = jnp.reshape(jnp.max(asv), (1, 1))
    bd = jnp.reshape(jnp.max(adv), (1, 1))

    @pl.when(i == 0)
    def _():
        ms_ref[...] = bs
        md_ref[...] = bd

    @pl.when(i > 0)
    def _():
        ms_ref[...] = jnp.maximum(ms_ref[...], bs)
        md_ref[...] = jnp.maximum(md_ref[...], bd)


_DENSE_OUT_SPECS = [
    pl.BlockSpec((2, _R, DH), lambda i: (0, i, 0)),
    pl.BlockSpec((_R, 1), lambda i: (i, 0)),
    pl.BlockSpec((_R, 1), lambda i: (i, 0)),
    pl.BlockSpec((1, 1), lambda i: (0, 0)),
    pl.BlockSpec((1, 1), lambda i: (0, 0)),
]
_DENSE_OUT_SHAPE = [
    jax.ShapeDtypeStruct((2, N_PAD, DH), _f32),
    jax.ShapeDtypeStruct((N_PAD, 1), _f32),
    jax.ShapeDtypeStruct((N_PAD, 1), _f32),
    jax.ShapeDtypeStruct((1, 1), _f32),
    jax.ShapeDtypeStruct((1, 1), _f32),
]


def _dense_first_body(x_ref, w_ref, avs_ref, avd_ref,
                      h_ref, as_ref, ad_ref, ms_ref, md_ref):
    xv = jnp.maximum(x_ref[...], 0.0)
    _dense_tail(xv, w_ref, avs_ref, avd_ref,
                h_ref, as_ref, ad_ref, ms_ref, md_ref)


def _dense_first(x, W, avs, avd):
    grid = (N_PAD // _R,)
    return pl.pallas_call(
        _dense_first_body,
        grid=grid,
        in_specs=[
            pl.BlockSpec((_R, D), lambda i: (i, 0)),
            pl.BlockSpec((D, D), lambda i: (0, 0)),
            pl.BlockSpec((1, D), lambda i: (0, 0)),
            pl.BlockSpec((1, D), lambda i: (0, 0)),
        ],
        out_specs=_DENSE_OUT_SPECS,
        out_shape=_DENSE_OUT_SHAPE,
    )(x, W, avs, avd)


def _merge_dense_body(acc_ref, den_ref, b_ref, w_ref, avs_ref, avd_ref,
                      h_ref, as_ref, ad_ref, ms_ref, md_ref):
    acc = jnp.concatenate([acc_ref[0], acc_ref[1]], axis=1)
    den = den_ref[...]
    x2 = jnp.maximum(acc / (den + 1e-16) + b_ref[...], 0.0)
    _dense_tail(x2, w_ref, avs_ref, avd_ref,
                h_ref, as_ref, ad_ref, ms_ref, md_ref)


def _merge_dense(acc, den, b, W, avs, avd):
    grid = (N_PAD // _R,)
    return pl.pallas_call(
        _merge_dense_body,
        grid=grid,
        in_specs=[
            pl.BlockSpec((2, _R, DH), lambda i: (0, i, 0)),
            pl.BlockSpec((_R, 1), lambda i: (i, 0)),
            pl.BlockSpec((1, D), lambda i: (0, 0)),
            pl.BlockSpec((D, D), lambda i: (0, 0)),
            pl.BlockSpec((1, D), lambda i: (0, 0)),
            pl.BlockSpec((1, D), lambda i: (0, 0)),
        ],
        out_specs=_DENSE_OUT_SPECS,
        out_shape=_DENSE_OUT_SHAPE,
    )(acc, den, b, W, avs, avd)


_RP = 1000


def _pool_body(acc_ref, den_ref, b_ref, bta_ref, btb_ref, wfc_ref, bfc_ref,
               out_ref, gmx, gsm, gct):
    i = pl.program_id(0)
    nblk = pl.num_programs(0)

    @pl.when(i == 0)
    def _():
        gmx[...] = jnp.zeros((G, D), _f32)
        gsm[...] = jnp.zeros((G, D), _f32)
        gct[...] = jnp.zeros((G, D), _f32)

    acc = jnp.concatenate([acc_ref[0], acc_ref[1]], axis=1)
    den = den_ref[...]
    x3 = jnp.maximum(acc / (den + 1e-16) + b_ref[...], 0.0)
    bb = bta_ref[0]
    btr = btb_ref[0]
    ioc = lax.broadcasted_iota(_i32, (G, _RP), 0)
    oh = (btr == ioc).astype(_f32)
    gsm[...] = gsm[...] + jax.lax.dot_general(
        oh, x3, (((1,), (0,)), ((), ())), preferred_element_type=_f32)
    gct[...] = gct[...] + jax.lax.dot_general(
        oh, jnp.ones_like(x3), (((1,), (0,)), ((), ())),
        preferred_element_type=_f32)
    g0 = jnp.min(bb)
    g1 = jnp.max(bb)

    def body(g, carry):
        mask = bb == g
        vals = jnp.where(mask, x3, 0.0)
        bm = jnp.max(vals, axis=0, keepdims=True)
        gmx[pl.ds(g, 1), :] = jnp.maximum(gmx[pl.ds(g, 1), :], bm)
        return carry

    lax.fori_loop(g0, g1 + 1, body, 0)

    @pl.when(i == nblk - 1)
    def _():
        cnt = jnp.maximum(gct[...], 1.0)
        gmean = gsm[...] / cnt
        pooled = jnp.concatenate([gmx[...], gmean], axis=1)
        out_ref[...] = jnp.dot(pooled, wfc_ref[...],
                               preferred_element_type=_f32) + bfc_ref[...]


def _pool(acc, den, b, bta, btb, Wfc, bfc):
    grid = (N_NODES // _RP,)
    return pl.pallas_call(
        _pool_body,
        grid=grid,
        in_specs=[
            pl.BlockSpec((2, _RP, DH), lambda i: (0, i, 0)),
            pl.BlockSpec((_RP, 1), lambda i: (i, 0)),
            pl.BlockSpec((1, D), lambda i: (0, 0)),
            pl.BlockSpec((1, _RP, 1), lambda i: (i, 0, 0)),
            pl.BlockSpec((1, 1, _RP), lambda i: (i, 0, 0)),
            pl.BlockSpec((2 * D, 1), lambda i: (0, 0)),
            pl.BlockSpec((1, 1), lambda i: (0, 0)),
        ],
        out_specs=pl.BlockSpec((G, 1), lambda i: (0, 0)),
        out_shape=jax.ShapeDtypeStruct((G, 1), _f32),
        scratch_shapes=[
            pltpu.VMEM((G, D), _f32),
            pltpu.VMEM((G, D), _f32),
            pltpu.VMEM((G, D), _f32),
        ],
    )(acc, den, b, bta, btb, Wfc, bfc)


def _edge_body(h_hbm, as_hbm, ad_hbm, src_hbm, dst_hbm, bnd_hbm,
               acc_out, den_out,
               as_v, ad_v, srcb, dstb, exb, exb2, exb3, exb4, exb5,
               rows, rows2, rows3, rows4, rows5, zb, bnd_v,
               acc_sc, den_sc, gs0, gs1, gs2, gs3, gs4,
               rs0, rs1, rs2, rs3, rs4, ds0, ds1, ds2, ds3, ds4):
    c = lax.axis_index("c")
    s = lax.axis_index("s")
    base_e = s * EPT
    row0 = pl.multiple_of(s * ROWS_PER_TILE, ROWS_PER_TILE)

    pltpu.sync_copy(as_hbm, as_v)
    pltpu.sync_copy(ad_hbm, ad_v)
    pltpu.sync_copy(bnd_hbm, bnd_v)
    b16 = bnd_v[...]
    zero16 = jnp.zeros((16,), _f32)
    iota16 = lax.iota(_i32, 16)

    def zr(r, carry):
        for j in range(DH // 16):
            rows[r, pl.ds(j * 16, 16)] = zero16
        return carry

    lax.fori_loop(0, 128, zr, 0)

    def zz(k, carry):
        zb[pl.ds(pl.multiple_of(k * 16, 16), 16)] = zero16
        return carry

    lax.fori_loop(0, ROWS_PER_TILE // 16, zz, 0)

    for k in range(ROWS_PER_TILE // 128):
        pltpu.sync_copy(rows, acc_sc.at[pl.ds(row0 + k * 128, 128)])
    pltpu.sync_copy(zb, den_sc.at[pl.ds(row0, ROWS_PER_TILE)])
    plsc.subcore_barrier()

    rows_bufs = (rows, rows2, rows3, rows4, rows5)
    ex_bufs = (exb, exb2, exb3, exb4, exb5)
    g_sems = (gs0, gs1, gs2, gs3, gs4)
    r_sems = (rs0, rs1, rs2, rs3, rs4)
    d_sems = (ds0, ds1, ds2, ds3, ds4)
    NBUF = 5
    LOOK = 2
    NCH_P = NCHUNK // 2

    for p in range(2):
        pltpu.sync_copy(src_hbm.at[s, pl.ds(p * NCH_P, NCH_P)], srcb)
        pltpu.sync_copy(dst_hbm.at[s, pl.ds(p * NCH_P, NCH_P)], dstb)

        for b in range(LOOK):
            pltpu.async_copy(
                h_hbm.at[c].at[srcb.at[b]], rows_bufs[b], g_sems[b])

        def pair_body(t, carry, _p=p):
            for b in range(NBUF):
                ch = NBUF * t + b
                rowsb, exbb = rows_bufs[b], ex_bufs[b]
                gsem, rsem, dsem = g_sems[b], r_sems[b], d_sems[b]
                ob = (b + LOOK) % NBUF
                orows, oexb = rows_bufs[ob], ex_bufs[ob]
                ogsem, orsem, odsem = g_sems[ob], r_sems[ob], d_sems[ob]

                @plsc.parallel_loop(0, 8, 1, unroll=4)
                def _scores(g):
                    sl = pl.ds(pl.multiple_of(g * 16, 16), 16)
                    src16 = srcb[ch, sl]
                    dst16 = dstb[ch, sl]
                    sg = plsc.load_gather(as_v, [src16])
                    dg = plsc.load_gather(ad_v, [dst16])
                    sv = sg + dg
                    ev = jnp.where(sv > 0.0, sv, 0.2 * sv)
                    ex = jnp.exp(ev - b16)
                    gidx = (base_e + (_p * NCH_P + ch) * 128
                            + g * 16 + iota16)
                    ex = jnp.where(gidx < N_EDGES, ex, 0.0)
                    exbb[0, sl] = ex

                @pl.when(ch + LOOK < NCH_P)
                def _():
                    @pl.when(ch >= NBUF - LOOK)
                    def _():
                        pltpu.make_async_copy(
                            orows, acc_sc.at[dstb.at[ch - (NBUF - LOOK)]],
                            orsem).wait()
                        pltpu.make_async_copy(
                            oexb.at[0],
                            den_sc.at[dstb.at[ch - (NBUF - LOOK)]],
                            odsem).wait()
                    pltpu.async_copy(
                        h_hbm.at[c].at[srcb.at[ch + LOOK]], orows, ogsem)

                pltpu.make_async_copy(
                    h_hbm.at[c].at[srcb.at[ch]], rowsb, gsem).wait()

                @plsc.parallel_loop(0, 128, 1, unroll=8)
                def _scale(e):
                    spl = plsc.load_gather(
                        exbb,
                        [jnp.zeros((16,), _i32), jnp.full((16,), e, _i32)])
                    for j in range(DH // 16):
                        slj = pl.ds(j * 16, 16)
                        rowsb[e, slj] = rowsb[e, slj] * spl

                pltpu.async_copy(rowsb, acc_sc.at[dstb.at[ch]], rsem,
                                 add=True)
                pltpu.async_copy(exbb.at[0], den_sc.at[dstb.at[ch]], dsem,
                                 add=True)
            return carry

        lax.fori_loop(0, NCH_P // NBUF, pair_body, 0)
        for b in range(NBUF):
            chl = NCH_P - NBUF + b
            bb = chl % NBUF
            pltpu.make_async_copy(
                rows_bufs[bb], acc_sc.at[dstb.at[chl]], r_sems[bb]).wait()
            pltpu.make_async_copy(
                ex_bufs[bb].at[0], den_sc.at[dstb.at[chl]],
                d_sems[bb]).wait()

    plsc.subcore_barrier()

    pltpu.sync_copy(acc_sc.at[pl.ds(row0, ROWS_PER_TILE)],
                    acc_out.at[c, pl.ds(row0, ROWS_PER_TILE)])

    @pl.when(c == 0)
    def _():
        pltpu.sync_copy(den_sc.at[pl.ds(row0, ROWS_PER_TILE)],
                        den_out.at[pl.ds(row0, ROWS_PER_TILE)])


def _edge(h_split, asv, adv, srcp, dstp, bnd16):
    mesh = plsc.VectorSubcoreMesh(core_axis_name="c", subcore_axis_name="s")
    kern = pl.kernel(
        _edge_body,
        out_type=[
            jax.ShapeDtypeStruct((2, N_PAD, DH), _f32),
            jax.ShapeDtypeStruct((N_PAD,), _f32),
        ],
        mesh=mesh,
        scratch_types=[
            pltpu.VMEM((N_PAD,), _f32),
            pltpu.VMEM((N_PAD,), _f32),
            pltpu.VMEM((NCHUNK // 2, 128), _i32),
            pltpu.VMEM((NCHUNK // 2, 128), _i32),
        ] + [pltpu.VMEM((1, 128), _f32)] * 5
          + [pltpu.VMEM((128, DH), _f32)] * 5
          + [
            pltpu.VMEM((ROWS_PER_TILE,), _f32),
            pltpu.VMEM((16,), _f32),
            pltpu.VMEM_SHARED((N_PAD, DH), _f32),
            pltpu.VMEM_SHARED((N_PAD,), _f32),
        ] + [pltpu.SemaphoreType.DMA] * 15,
        compiler_params=pltpu.CompilerParams(
            needs_layout_passes=False, use_tc_tiling_on_sc=False),
    )
    return kern(h_split, asv, adv, srcp, dstp, bnd16)


def _bound_scalar(ms, md):
    b = ms[0, 0] + md[0, 0]
    b = jnp.where(b > 0, b, 0.2 * b)
    return jnp.full((16,), b, _f32)


def kernel(x, edge_index, batch, W1, a1s, a1d, b1, W2, a2s, a2d, b2, Wfc, bfc):
    src = edge_index[0].astype(_i32)
    dst = edge_index[1].astype(_i32)
    padlen = E_PAD - N_EDGES
    srcp = jnp.concatenate([src, jnp.zeros((padlen,), _i32)]).reshape(
        NSUB, NCHUNK, 128)
    dstp = jnp.concatenate([dst, jnp.zeros((padlen,), _i32)]).reshape(
        NSUB, NCHUNK, 128)
    xpad = jnp.pad(x, ((0, N_PAD - N_NODES), (0, 0)))
    bt = batch.astype(_i32)
    bta = bt.reshape(N_NODES // _RP, _RP, 1)
    btb = bt.reshape(N_NODES // _RP, 1, _RP)

    h1, as1, ad1, ms1, md1 = _dense_first(
        xpad, W1, a1s.reshape(1, D), a1d.reshape(1, D))
    acc1, den1 = _edge(h1, as1[:, 0], ad1[:, 0], srcp, dstp,
                       _bound_scalar(ms1, md1))
    h2, as2, ad2, ms2, md2 = _merge_dense(
        acc1, den1.reshape(N_PAD, 1), b1.reshape(1, D), W2,
        a2s.reshape(1, D), a2d.reshape(1, D))
    acc2, den2 = _edge(h2, as2[:, 0], ad2[:, 0], srcp, dstp,
                       _bound_scalar(ms2, md2))
    out = _pool(acc2[:, :N_NODES], den2[:N_NODES].reshape(N_NODES, 1),
                b2.reshape(1, D), bta, btb, Wfc, bfc.reshape(1, 1))
    return out


# speedup vs baseline: 1.0047x; 1.0047x over previous
import functools

import jax
import jax.numpy as jnp
from jax import lax
from jax.experimental import pallas as pl
from jax.experimental.pallas import tpu as pltpu
from jax.experimental.pallas import tpu_sc as plsc

N_NODES = 10000
N_PAD = 10240
N_EDGES = 320000
E_PAD = 327680
D = 128
G = 64
DH = 64
NSUB = 16
EPT = E_PAD // NSUB
NCHUNK = EPT // 128
ROWS_PER_TILE = N_PAD // NSUB

_f32 = jnp.float32
_i32 = jnp.int32


_R = 1024


def _dense_tail(xv, w_ref, avs_ref, avd_ref,
                h_ref, as_ref, ad_ref, ms_ref, md_ref):
    i = pl.program_id(0)
    h = jnp.dot(xv, w_ref[...], preferred_element_type=_f32)
    h_ref[0] = h[:, :DH]
    h_ref[1] = h[:, DH:]
    asv = jnp.sum(h * avs_ref[...], axis=1, keepdims=True)
    adv = jnp.sum(h * avd_ref[...], axis=1, keepdims=True)
    as_ref[...] = asv
    ad_ref[...] = adv
    bs = jnp.reshape(jnp.max(asv), (1, 1))
    bd = jnp.reshape(jnp.max(adv), (1, 1))

    @pl.when(i == 0)
    def _():
        ms_ref[...] = bs
        md_ref[...] = bd

    @pl.when(i > 0)
    def _():
        ms_ref[...] = jnp.maximum(ms_ref[...], bs)
        md_ref[...] = jnp.maximum(md_ref[...], bd)


_DENSE_OUT_SPECS = [
    pl.BlockSpec((2, _R, DH), lambda i: (0, i, 0)),
    pl.BlockSpec((_R, 1), lambda i: (i, 0)),
    pl.BlockSpec((_R, 1), lambda i: (i, 0)),
    pl.BlockSpec((1, 1), lambda i: (0, 0)),
    pl.BlockSpec((1, 1), lambda i: (0, 0)),
]
_DENSE_OUT_SHAPE = [
    jax.ShapeDtypeStruct((2, N_PAD, DH), _f32),
    jax.ShapeDtypeStruct((N_PAD, 1), _f32),
    jax.ShapeDtypeStruct((N_PAD, 1), _f32),
    jax.ShapeDtypeStruct((1, 1), _f32),
    jax.ShapeDtypeStruct((1, 1), _f32),
]


def _dense_first_body(x_ref, w_ref, avs_ref, avd_ref,
                      h_ref, as_ref, ad_ref, ms_ref, md_ref):
    xv = jnp.maximum(x_ref[...], 0.0)
    _dense_tail(xv, w_ref, avs_ref, avd_ref,
                h_ref, as_ref, ad_ref, ms_ref, md_ref)


def _dense_first(x, W, avs, avd):
    grid = (N_PAD // _R,)
    return pl.pallas_call(
        _dense_first_body,
        grid=grid,
        in_specs=[
            pl.BlockSpec((_R, D), lambda i: (i, 0)),
            pl.BlockSpec((D, D), lambda i: (0, 0)),
            pl.BlockSpec((1, D), lambda i: (0, 0)),
            pl.BlockSpec((1, D), lambda i: (0, 0)),
        ],
        out_specs=_DENSE_OUT_SPECS,
        out_shape=_DENSE_OUT_SHAPE,
    )(x, W, avs, avd)


def _merge_dense_body(acc_ref, den_ref, b_ref, w_ref, avs_ref, avd_ref,
                      h_ref, as_ref, ad_ref, ms_ref, md_ref):
    acc = jnp.concatenate([acc_ref[0], acc_ref[1]], axis=1)
    den = den_ref[...]
    x2 = jnp.maximum(acc / (den + 1e-16) + b_ref[...], 0.0)
    _dense_tail(x2, w_ref, avs_ref, avd_ref,
                h_ref, as_ref, ad_ref, ms_ref, md_ref)


def _merge_dense(acc, den, b, W, avs, avd):
    grid = (N_PAD // _R,)
    return pl.pallas_call(
        _merge_dense_body,
        grid=grid,
        in_specs=[
            pl.BlockSpec((2, _R, DH), lambda i: (0, i, 0)),
            pl.BlockSpec((_R, 1), lambda i: (i, 0)),
            pl.BlockSpec((1, D), lambda i: (0, 0)),
            pl.BlockSpec((D, D), lambda i: (0, 0)),
            pl.BlockSpec((1, D), lambda i: (0, 0)),
            pl.BlockSpec((1, D), lambda i: (0, 0)),
        ],
        out_specs=_DENSE_OUT_SPECS,
        out_shape=_DENSE_OUT_SHAPE,
    )(acc, den, b, W, avs, avd)


_RP = 1000


def _pool_body(acc_ref, den_ref, b_ref, bta_ref, btb_ref, wfc_ref, bfc_ref,
               out_ref, gmx, gsm, gct):
    i = pl.program_id(0)
    nblk = pl.num_programs(0)

    @pl.when(i == 0)
    def _():
        gmx[...] = jnp.zeros((G, D), _f32)
        gsm[...] = jnp.zeros((G, D), _f32)
        gct[...] = jnp.zeros((G, D), _f32)

    acc = jnp.concatenate([acc_ref[0], acc_ref[1]], axis=1)
    den = den_ref[...]
    x3 = jnp.maximum(acc / (den + 1e-16) + b_ref[...], 0.0)
    bb = bta_ref[0]
    btr = btb_ref[0]
    ioc = lax.broadcasted_iota(_i32, (G, _RP), 0)
    oh = (btr == ioc).astype(_f32)
    gsm[...] = gsm[...] + jax.lax.dot_general(
        oh, x3, (((1,), (0,)), ((), ())), preferred_element_type=_f32)
    gct[...] = gct[...] + jax.lax.dot_general(
        oh, jnp.ones_like(x3), (((1,), (0,)), ((), ())),
        preferred_element_type=_f32)
    g0 = jnp.min(bb)
    g1 = jnp.max(bb)

    def body(g, carry):
        mask = bb == g
        vals = jnp.where(mask, x3, 0.0)
        bm = jnp.max(vals, axis=0, keepdims=True)
        gmx[pl.ds(g, 1), :] = jnp.maximum(gmx[pl.ds(g, 1), :], bm)
        return carry

    lax.fori_loop(g0, g1 + 1, body, 0)

    @pl.when(i == nblk - 1)
    def _():
        cnt = jnp.maximum(gct[...], 1.0)
        gmean = gsm[...] / cnt
        pooled = jnp.concatenate([gmx[...], gmean], axis=1)
        out_ref[...] = jnp.dot(pooled, wfc_ref[...],
                               preferred_element_type=_f32) + bfc_ref[...]


def _pool(acc, den, b, bta, btb, Wfc, bfc):
    grid = (N_NODES // _RP,)
    return pl.pallas_call(
        _pool_body,
        grid=grid,
        in_specs=[
            pl.BlockSpec((2, _RP, DH), lambda i: (0, i, 0)),
            pl.BlockSpec((_RP, 1), lambda i: (i, 0)),
            pl.BlockSpec((1, D), lambda i: (0, 0)),
            pl.BlockSpec((1, _RP, 1), lambda i: (i, 0, 0)),
            pl.BlockSpec((1, 1, _RP), lambda i: (i, 0, 0)),
            pl.BlockSpec((2 * D, 1), lambda i: (0, 0)),
            pl.BlockSpec((1, 1), lambda i: (0, 0)),
        ],
        out_specs=pl.BlockSpec((G, 1), lambda i: (0, 0)),
        out_shape=jax.ShapeDtypeStruct((G, 1), _f32),
        scratch_shapes=[
            pltpu.VMEM((G, D), _f32),
            pltpu.VMEM((G, D), _f32),
            pltpu.VMEM((G, D), _f32),
        ],
    )(acc, den, b, bta, btb, Wfc, bfc)


def _edge_body(h_hbm, as_hbm, ad_hbm, src_hbm, dst_hbm, bnd_hbm,
               acc_out, den_out,
               as_v, ad_v, srcb, dstb, exb, exb2, exb3, exb4, exb5,
               rows, rows2, rows3, rows4, rows5, zb, bnd_v,
               acc_sc, den_sc, gs0, gs1, gs2, gs3, gs4,
               rs0, rs1, rs2, rs3, rs4, ds0, ds1, ds2, ds3, ds4):
    c = lax.axis_index("c")
    s = lax.axis_index("s")
    base_e = s * EPT
    row0 = pl.multiple_of(s * ROWS_PER_TILE, ROWS_PER_TILE)

    pltpu.sync_copy(as_hbm, as_v)
    pltpu.sync_copy(ad_hbm, ad_v)
    pltpu.sync_copy(bnd_hbm, bnd_v)
    b16 = bnd_v[...]
    zero16 = jnp.zeros((16,), _f32)
    iota16 = lax.iota(_i32, 16)

    def zr(r, carry):
        for j in range(DH // 16):
            rows[r, pl.ds(j * 16, 16)] = zero16
        return carry

    lax.fori_loop(0, 128, zr, 0)

    def zz(k, carry):
        zb[pl.ds(pl.multiple_of(k * 16, 16), 16)] = zero16
        return carry

    lax.fori_loop(0, ROWS_PER_TILE // 16, zz, 0)

    for k in range(ROWS_PER_TILE // 128):
        pltpu.sync_copy(rows, acc_sc.at[pl.ds(row0 + k * 128, 128)])
    pltpu.sync_copy(zb, den_sc.at[pl.ds(row0, ROWS_PER_TILE)])
    plsc.subcore_barrier()

    rows_bufs = (rows, rows2, rows3, rows4, rows5)
    ex_bufs = (exb, exb2, exb3, exb4, exb5)
    g_sems = (gs0, gs1, gs2, gs3, gs4)
    r_sems = (rs0, rs1, rs2, rs3, rs4)
    d_sems = (ds0, ds1, ds2, ds3, ds4)
    NBUF = 5
    LOOK = 2
    NCH_P = NCHUNK // 2

    for p in range(2):
        pltpu.sync_copy(src_hbm.at[s, pl.ds(p * NCH_P, NCH_P)], srcb)
        pltpu.sync_copy(dst_hbm.at[s, pl.ds(p * NCH_P, NCH_P)], dstb)

        for b in range(LOOK):
            pltpu.async_copy(
                h_hbm.at[c].at[srcb.at[b]], rows_bufs[b], g_sems[b])

        def pair_body(t, carry, _p=p):
            for b in range(NBUF):
                ch = NBUF * t + b
                rowsb, exbb = rows_bufs[b], ex_bufs[b]
                gsem, rsem, dsem = g_sems[b], r_sems[b], d_sems[b]
                ob = (b + LOOK) % NBUF
                orows, oexb = rows_bufs[ob], ex_bufs[ob]
                ogsem, orsem, odsem = g_sems[ob], r_sems[ob], d_sems[ob]

                @plsc.parallel_loop(0, 8, 1, unroll=8)
                def _scores(g):
                    sl = pl.ds(pl.multiple_of(g * 16, 16), 16)
                    src16 = srcb[ch, sl]
                    dst16 = dstb[ch, sl]
                    sg = plsc.load_gather(as_v, [src16])
                    dg = plsc.load_gather(ad_v, [dst16])
                    sv = sg + dg
                    ev = jnp.where(sv > 0.0, sv, 0.2 * sv)
                    ex = jnp.exp(ev - b16)
                    gidx = (base_e + (_p * NCH_P + ch) * 128
                            + g * 16 + iota16)
                    ex = jnp.where(gidx < N_EDGES, ex, 0.0)
                    exbb[0, sl] = ex

                @pl.when(ch + LOOK < NCH_P)
                def _():
                    @pl.when(ch >= NBUF - LOOK)
                    def _():
                        pltpu.make_async_copy(
                            orows, acc_sc.at[dstb.at[ch - (NBUF - LOOK)]],
                            orsem).wait()
                        pltpu.make_async_copy(
                            oexb.at[0],
                            den_sc.at[dstb.at[ch - (NBUF - LOOK)]],
                            odsem).wait()
                    pltpu.async_copy(
                        h_hbm.at[c].at[srcb.at[ch + LOOK]], orows, ogsem)

                pltpu.make_async_copy(
                    h_hbm.at[c].at[srcb.at[ch]], rowsb, gsem).wait()

                @plsc.parallel_loop(0, 128, 1, unroll=16)
                def _scale(e):
                    spl = plsc.load_gather(
                        exbb,
                        [jnp.zeros((16,), _i32), jnp.full((16,), e, _i32)])
                    for j in range(DH // 16):
                        slj = pl.ds(j * 16, 16)
                        rowsb[e, slj] = rowsb[e, slj] * spl

                pltpu.async_copy(rowsb, acc_sc.at[dstb.at[ch]], rsem,
                                 add=True)
                pltpu.async_copy(exbb.at[0], den_sc.at[dstb.at[ch]], dsem,
                                 add=True)
            return carry

        lax.fori_loop(0, NCH_P // NBUF, pair_body, 0)
        for b in range(NBUF):
            chl = NCH_P - NBUF + b
            bb = chl % NBUF
            pltpu.make_async_copy(
                rows_bufs[bb], acc_sc.at[dstb.at[chl]], r_sems[bb]).wait()
            pltpu.make_async_copy(
                ex_bufs[bb].at[0], den_sc.at[dstb.at[chl]],
                d_sems[bb]).wait()

    plsc.subcore_barrier()

    pltpu.sync_copy(acc_sc.at[pl.ds(row0, ROWS_PER_TILE)],
                    acc_out.at[c, pl.ds(row0, ROWS_PER_TILE)])

    @pl.when(c == 0)
    def _():
        pltpu.sync_copy(den_sc.at[pl.ds(row0, ROWS_PER_TILE)],
                        den_out.at[pl.ds(row0, ROWS_PER_TILE)])


def _edge(h_split, asv, adv, srcp, dstp, bnd16):
    mesh = plsc.VectorSubcoreMesh(core_axis_name="c", subcore_axis_name="s")
    kern = pl.kernel(
        _edge_body,
        out_type=[
            jax.ShapeDtypeStruct((2, N_PAD, DH), _f32),
            jax.ShapeDtypeStruct((N_PAD,), _f32),
        ],
        mesh=mesh,
        scratch_types=[
            pltpu.VMEM((N_PAD,), _f32),
            pltpu.VMEM((N_PAD,), _f32),
            pltpu.VMEM((NCHUNK // 2, 128), _i32),
            pltpu.VMEM((NCHUNK // 2, 128), _i32),
        ] + [pltpu.VMEM((1, 128), _f32)] * 5
          + [pltpu.VMEM((128, DH), _f32)] * 5
          + [
            pltpu.VMEM((ROWS_PER_TILE,), _f32),
            pltpu.VMEM((16,), _f32),
            pltpu.VMEM_SHARED((N_PAD, DH), _f32),
            pltpu.VMEM_SHARED((N_PAD,), _f32),
        ] + [pltpu.SemaphoreType.DMA] * 15,
        compiler_params=pltpu.CompilerParams(
            needs_layout_passes=False, use_tc_tiling_on_sc=False),
    )
    return kern(h_split, asv, adv, srcp, dstp, bnd16)


def _bound_scalar(ms, md):
    b = ms[0, 0] + md[0, 0]
    b = jnp.where(b > 0, b, 0.2 * b)
    return jnp.full((16,), b, _f32)


def kernel(x, edge_index, batch, W1, a1s, a1d, b1, W2, a2s, a2d, b2, Wfc, bfc):
    src = edge_index[0].astype(_i32)
    dst = edge_index[1].astype(_i32)
    padlen = E_PAD - N_EDGES
    srcp = jnp.concatenate([src, jnp.zeros((padlen,), _i32)]).reshape(
        NSUB, NCHUNK, 128)
    dstp = jnp.concatenate([dst, jnp.zeros((padlen,), _i32)]).reshape(
        NSUB, NCHUNK, 128)
    xpad = jnp.pad(x, ((0, N_PAD - N_NODES), (0, 0)))
    bt = batch.astype(_i32)
    bta = bt.reshape(N_NODES // _RP, _RP, 1)
    btb = bt.reshape(N_NODES // _RP, 1, _RP)

    h1, as1, ad1, ms1, md1 = _dense_first(
        xpad, W1, a1s.reshape(1, D), a1d.reshape(1, D))
    acc1, den1 = _edge(h1, as1[:, 0], ad1[:, 0], srcp, dstp,
                       _bound_scalar(ms1, md1))
    h2, as2, ad2, ms2, md2 = _merge_dense(
        acc1, den1.reshape(N_PAD, 1), b1.reshape(1, D), W2,
        a2s.reshape(1, D), a2d.reshape(1, D))
    acc2, den2 = _edge(h2, as2[:, 0], ad2[:, 0], srcp, dstp,
                       _bound_scalar(ms2, md2))
    out = _pool(acc2[:, :N_NODES], den2[:N_NODES].reshape(N_NODES, 1),
                b2.reshape(1, D), bta, btb, Wfc, bfc.reshape(1, 1))
    return out


# speedup vs baseline: 1.3282x; 1.3220x over previous
import functools

import jax
import jax.numpy as jnp
from jax import lax
from jax.experimental import pallas as pl
from jax.experimental.pallas import tpu as pltpu
from jax.experimental.pallas import tpu_sc as plsc

N_NODES = 10000
N_PAD = 10240
N_EDGES = 320000
E_PAD = 327680
D = 128
G = 64
DH = 64
NSUB = 16
EPT = E_PAD // NSUB
NCHUNK = EPT // 128
ROWS_PER_TILE = N_PAD // NSUB

_f32 = jnp.float32
_i32 = jnp.int32
_HIMASK = -65536


_R = 1024


def _dense_tail(xv, w_ref, avs_ref, avd_ref,
                h_ref, as_ref, ad_ref, ms_ref, md_ref):
    i = pl.program_id(0)
    h = jnp.dot(xv, w_ref[...], preferred_element_type=_f32)
    h_ref[0] = h[:, :DH]
    h_ref[1] = h[:, DH:]
    asv = jnp.sum(h * avs_ref[...], axis=1, keepdims=True)
    adv = jnp.sum(h * avd_ref[...], axis=1, keepdims=True)
    as_ref[...] = asv
    ad_ref[...] = adv
    bs = jnp.reshape(jnp.max(asv), (1, 1))
    bd = jnp.reshape(jnp.max(adv), (1, 1))

    @pl.when(i == 0)
    def _():
        ms_ref[...] = bs
        md_ref[...] = bd

    @pl.when(i > 0)
    def _():
        ms_ref[...] = jnp.maximum(ms_ref[...], bs)
        md_ref[...] = jnp.maximum(md_ref[...], bd)


_DENSE_OUT_SPECS = [
    pl.BlockSpec((2, _R, DH), lambda i: (0, i, 0)),
    pl.BlockSpec((_R, 1), lambda i: (i, 0)),
    pl.BlockSpec((_R, 1), lambda i: (i, 0)),
    pl.BlockSpec((1, 1), lambda i: (0, 0)),
    pl.BlockSpec((1, 1), lambda i: (0, 0)),
]
_DENSE_OUT_SHAPE = [
    jax.ShapeDtypeStruct((2, N_PAD, DH), _f32),
    jax.ShapeDtypeStruct((N_PAD, 1), _f32),
    jax.ShapeDtypeStruct((N_PAD, 1), _f32),
    jax.ShapeDtypeStruct((1, 1), _f32),
    jax.ShapeDtypeStruct((1, 1), _f32),
]


def _dense_first_body(x_ref, w_ref, avs_ref, avd_ref,
                      h_ref, as_ref, ad_ref, ms_ref, md_ref):
    xv = jnp.maximum(x_ref[...], 0.0)
    _dense_tail(xv, w_ref, avs_ref, avd_ref,
                h_ref, as_ref, ad_ref, ms_ref, md_ref)


def _dense_first(x, W, avs, avd):
    grid = (N_PAD // _R,)
    return pl.pallas_call(
        _dense_first_body,
        grid=grid,
        in_specs=[
            pl.BlockSpec((_R, D), lambda i: (i, 0)),
            pl.BlockSpec((D, D), lambda i: (0, 0)),
            pl.BlockSpec((1, D), lambda i: (0, 0)),
            pl.BlockSpec((1, D), lambda i: (0, 0)),
        ],
        out_specs=_DENSE_OUT_SPECS,
        out_shape=_DENSE_OUT_SHAPE,
    )(x, W, avs, avd)


def _merge_dense_body(acc_ref, den_ref, b_ref, w_ref, avs_ref, avd_ref,
                      h_ref, as_ref, ad_ref, ms_ref, md_ref):
    acc = jnp.concatenate([acc_ref[0], acc_ref[1]], axis=1)
    den = den_ref[...]
    x2 = jnp.maximum(acc / (den + 1e-16) + b_ref[...], 0.0)
    _dense_tail(x2, w_ref, avs_ref, avd_ref,
                h_ref, as_ref, ad_ref, ms_ref, md_ref)


def _merge_dense(acc, den, b, W, avs, avd):
    grid = (N_PAD // _R,)
    return pl.pallas_call(
        _merge_dense_body,
        grid=grid,
        in_specs=[
            pl.BlockSpec((2, _R, DH), lambda i: (0, i, 0)),
            pl.BlockSpec((_R, 1), lambda i: (i, 0)),
            pl.BlockSpec((1, D), lambda i: (0, 0)),
            pl.BlockSpec((D, D), lambda i: (0, 0)),
            pl.BlockSpec((1, D), lambda i: (0, 0)),
            pl.BlockSpec((1, D), lambda i: (0, 0)),
        ],
        out_specs=_DENSE_OUT_SPECS,
        out_shape=_DENSE_OUT_SHAPE,
    )(acc, den, b, W, avs, avd)


_RP = 1000


def _pool_body(acc_ref, den_ref, b_ref, bta_ref, btb_ref, wfc_ref, bfc_ref,
               out_ref, gmx, gsm, gct):
    i = pl.program_id(0)
    nblk = pl.num_programs(0)

    @pl.when(i == 0)
    def _():
        gmx[...] = jnp.zeros((G, D), _f32)
        gsm[...] = jnp.zeros((G, D), _f32)
        gct[...] = jnp.zeros((G, D), _f32)

    acc = jnp.concatenate([acc_ref[0], acc_ref[1]], axis=1)
    den = den_ref[...]
    x3 = jnp.maximum(acc / (den + 1e-16) + b_ref[...], 0.0)
    bb = bta_ref[0]
    btr = btb_ref[0]
    ioc = lax.broadcasted_iota(_i32, (G, _RP), 0)
    oh = (btr == ioc).astype(_f32)
    gsm[...] = gsm[...] + jax.lax.dot_general(
        oh, x3, (((1,), (0,)), ((), ())), preferred_element_type=_f32)
    gct[...] = gct[...] + jax.lax.dot_general(
        oh, jnp.ones_like(x3), (((1,), (0,)), ((), ())),
        preferred_element_type=_f32)
    g0 = jnp.min(bb)
    g1 = jnp.max(bb)

    def body(g, carry):
        mask = bb == g
        vals = jnp.where(mask, x3, 0.0)
        bm = jnp.max(vals, axis=0, keepdims=True)
        gmx[pl.ds(g, 1), :] = jnp.maximum(gmx[pl.ds(g, 1), :], bm)
        return carry

    lax.fori_loop(g0, g1 + 1, body, 0)

    @pl.when(i == nblk - 1)
    def _():
        cnt = jnp.maximum(gct[...], 1.0)
        gmean = gsm[...] / cnt
        pooled = jnp.concatenate([gmx[...], gmean], axis=1)
        out_ref[...] = jnp.dot(pooled, wfc_ref[...],
                               preferred_element_type=_f32) + bfc_ref[...]


def _pool(acc, den, b, bta, btb, Wfc, bfc):
    grid = (N_NODES // _RP,)
    return pl.pallas_call(
        _pool_body,
        grid=grid,
        in_specs=[
            pl.BlockSpec((2, _RP, DH), lambda i: (0, i, 0)),
            pl.BlockSpec((_RP, 1), lambda i: (i, 0)),
            pl.BlockSpec((1, D), lambda i: (0, 0)),
            pl.BlockSpec((1, _RP, 1), lambda i: (i, 0, 0)),
            pl.BlockSpec((1, 1, _RP), lambda i: (i, 0, 0)),
            pl.BlockSpec((2 * D, 1), lambda i: (0, 0)),
            pl.BlockSpec((1, 1), lambda i: (0, 0)),
        ],
        out_specs=pl.BlockSpec((G, 1), lambda i: (0, 0)),
        out_shape=jax.ShapeDtypeStruct((G, 1), _f32),
        scratch_shapes=[
            pltpu.VMEM((G, D), _f32),
            pltpu.VMEM((G, D), _f32),
            pltpu.VMEM((G, D), _f32),
        ],
    )(acc, den, b, bta, btb, Wfc, bfc)


def _edge_body(h_hbm, as_hbm, ad_hbm, src_hbm, dst_hbm, bnd_hbm,
               acc_out, den_out,
               as_v, ad_v, srcb, dstb, exb, exb2, exb3, exb4, exb5,
               rp1, rp2, rp3, rp4, rp5,
               rows, rows2, rows3, rows4, rows5, zb, bnd_v,
               acc_sc, den_sc, gs0, gs1, gs2, gs3, gs4,
               rs0, rs1, rs2, rs3, rs4, ds0, ds1, ds2, ds3, ds4):
    c = lax.axis_index("c")
    s = lax.axis_index("s")
    base_e = s * EPT
    row0 = pl.multiple_of(s * ROWS_PER_TILE, ROWS_PER_TILE)

    pltpu.sync_copy(as_hbm, as_v)
    pltpu.sync_copy(ad_hbm, ad_v)
    pltpu.sync_copy(bnd_hbm, bnd_v)
    b16 = bnd_v[...]
    zero16 = jnp.zeros((16,), _f32)
    iota16 = lax.iota(_i32, 16)

    def zr(r, carry):
        for j in range(DH // 16):
            rows[r, pl.ds(j * 16, 16)] = zero16
        return carry

    lax.fori_loop(0, 128, zr, 0)

    def zz(k, carry):
        zb[pl.ds(pl.multiple_of(k * 16, 16), 16)] = zero16
        return carry

    lax.fori_loop(0, ROWS_PER_TILE // 16, zz, 0)

    for k in range(ROWS_PER_TILE // 128):
        pltpu.sync_copy(rows, acc_sc.at[pl.ds(row0 + k * 128, 128)])
    pltpu.sync_copy(zb, den_sc.at[pl.ds(row0, ROWS_PER_TILE)])
    plsc.subcore_barrier()

    rows_bufs = (rows, rows2, rows3, rows4, rows5)
    ex_bufs = (exb, exb2, exb3, exb4, exb5)
    g_sems = (gs0, gs1, gs2, gs3, gs4)
    r_sems = (rs0, rs1, rs2, rs3, rs4)
    d_sems = (ds0, ds1, ds2, ds3, ds4)
    rp_bufs = (rp1, rp2, rp3, rp4, rp5)
    NBUF = 5
    LOOK = 2
    NCH_P = NCHUNK // 8

    def pass_body(p, pcarry):
        pltpu.sync_copy(src_hbm.at[s, pl.ds(p * NCH_P, NCH_P)], srcb)
        pltpu.sync_copy(dst_hbm.at[s, pl.ds(p * NCH_P, NCH_P)], dstb)

        for b in range(LOOK):
            pltpu.async_copy(
                h_hbm.at[c].at[srcb.at[b]], rp_bufs[b], g_sems[b])

        def pair_body(t, carry):
            for b in range(NBUF):
                ch = NBUF * t + b
                rowsb, rpb, exbb = rows_bufs[b], rp_bufs[b], ex_bufs[b]
                gsem, rsem, dsem = g_sems[b], r_sems[b], d_sems[b]
                ob = (b + LOOK) % NBUF
                orows, orp, oexb = rows_bufs[ob], rp_bufs[ob], ex_bufs[ob]
                ogsem, orsem, odsem = g_sems[ob], r_sems[ob], d_sems[ob]

                @plsc.parallel_loop(0, 8, 1, unroll=8)
                def _scores(g):
                    sl = pl.ds(pl.multiple_of(g * 16, 16), 16)
                    src16 = srcb[ch, sl]
                    dst16 = dstb[ch, sl]
                    sg = plsc.load_gather(as_v, [src16])
                    dg = plsc.load_gather(ad_v, [dst16])
                    sv = sg + dg
                    ev = jnp.where(sv > 0.0, sv, 0.2 * sv)
                    ex = jnp.exp(ev - b16)
                    gidx = (base_e + (p * NCH_P + ch) * 128
                            + g * 16 + iota16)
                    ex = jnp.where(gidx < N_EDGES, ex, 0.0)
                    exbb[0, sl] = ex

                @pl.when(ch + LOOK < NCH_P)
                def _():
                    @pl.when(ch >= NBUF - LOOK)
                    def _():
                        pltpu.make_async_copy(
                            orows, acc_sc.at[dstb.at[ch - (NBUF - LOOK)]],
                            orsem).wait()
                        pltpu.make_async_copy(
                            oexb.at[0],
                            den_sc.at[dstb.at[ch - (NBUF - LOOK)]],
                            odsem).wait()
                    pltpu.async_copy(
                        h_hbm.at[c].at[srcb.at[ch + LOOK]], orp, ogsem)

                pltpu.make_async_copy(
                    h_hbm.at[c].at[srcb.at[ch]], rpb, gsem).wait()

                @plsc.parallel_loop(0, 128, 1, unroll=8)
                def _scale(e):
                    spl = plsc.load_gather(
                        exbb,
                        [jnp.zeros((16,), _i32), jnp.full((16,), e, _i32)])
                    for g in range(2):
                        vi = rpb[e, pl.ds(g * 16, 16)]
                        fa = plsc.bitcast(vi << 16, _f32) * spl
                        fb = plsc.bitcast(vi & _HIMASK, _f32) * spl
                        rowsb[e, pl.ds(g * 32, 16)] = fa
                        rowsb[e, pl.ds(g * 32 + 16, 16)] = fb

                pltpu.async_copy(rowsb, acc_sc.at[dstb.at[ch]], rsem,
                                 add=True)
                pltpu.async_copy(exbb.at[0], den_sc.at[dstb.at[ch]], dsem,
                                 add=True)
            return carry

        lax.fori_loop(0, NCH_P // NBUF, pair_body, 0)
        for b in range(NBUF):
            chl = NCH_P - NBUF + b
            bb = chl % NBUF
            pltpu.make_async_copy(
                rows_bufs[bb], acc_sc.at[dstb.at[chl]], r_sems[bb]).wait()
            pltpu.make_async_copy(
                ex_bufs[bb].at[0], den_sc.at[dstb.at[chl]],
                d_sems[bb]).wait()
        return pcarry

    lax.fori_loop(0, 8, pass_body, 0)
    plsc.subcore_barrier()

    pltpu.sync_copy(acc_sc.at[pl.ds(row0, ROWS_PER_TILE)],
                    acc_out.at[c, pl.ds(row0, ROWS_PER_TILE)])

    @pl.when(c == 0)
    def _():
        pltpu.sync_copy(den_sc.at[pl.ds(row0, ROWS_PER_TILE)],
                        den_out.at[pl.ds(row0, ROWS_PER_TILE)])


def _edge(h_packed, asv, adv, srcp, dstp, bnd16):
    mesh = plsc.VectorSubcoreMesh(core_axis_name="c", subcore_axis_name="s")
    kern = pl.kernel(
        _edge_body,
        out_type=[
            jax.ShapeDtypeStruct((2, N_PAD, DH), _f32),
            jax.ShapeDtypeStruct((N_PAD,), _f32),
        ],
        mesh=mesh,
        scratch_types=[
            pltpu.VMEM((N_PAD,), _f32),
            pltpu.VMEM((N_PAD,), _f32),
            pltpu.VMEM((NCHUNK // 8, 128), _i32),
            pltpu.VMEM((NCHUNK // 8, 128), _i32),
        ] + [pltpu.VMEM((1, 128), _f32)] * 5
          + [pltpu.VMEM((128, DH // 2), _i32)] * 5
          + [pltpu.VMEM((128, DH), _f32)] * 5
          + [
            pltpu.VMEM((ROWS_PER_TILE,), _f32),
            pltpu.VMEM((16,), _f32),
            pltpu.VMEM_SHARED((N_PAD, DH), _f32),
            pltpu.VMEM_SHARED((N_PAD,), _f32),
        ] + [pltpu.SemaphoreType.DMA] * 15,
        compiler_params=pltpu.CompilerParams(
            needs_layout_passes=False, use_tc_tiling_on_sc=False),
    )
    return kern(h_packed, asv, adv, srcp, dstp, bnd16)


def _pack_rows(h):
    hg = h.reshape(2, N_PAD, 2, 2, 16)
    hper = jnp.transpose(hg, (0, 1, 2, 4, 3)).astype(jnp.bfloat16)
    return jax.lax.bitcast_convert_type(hper, _i32).reshape(2, N_PAD, DH // 2)


def _bound_scalar(ms, md):
    b = ms[0, 0] + md[0, 0]
    b = jnp.where(b > 0, b, 0.2 * b)
    return jnp.full((16,), b, _f32)


def kernel(x, edge_index, batch, W1, a1s, a1d, b1, W2, a2s, a2d, b2, Wfc, bfc):
    src = edge_index[0].astype(_i32)
    dst = edge_index[1].astype(_i32)
    padlen = E_PAD - N_EDGES
    srcp = jnp.concatenate([src, jnp.zeros((padlen,), _i32)]).reshape(
        NSUB, NCHUNK, 128)
    dstp = jnp.concatenate([dst, jnp.zeros((padlen,), _i32)]).reshape(
        NSUB, NCHUNK, 128)
    xpad = jnp.pad(x, ((0, N_PAD - N_NODES), (0, 0)))
    bt = batch.astype(_i32)
    bta = bt.reshape(N_NODES // _RP, _RP, 1)
    btb = bt.reshape(N_NODES // _RP, 1, _RP)

    h1, as1, ad1, ms1, md1 = _dense_first(
        xpad, W1, a1s.reshape(1, D), a1d.reshape(1, D))
    acc1, den1 = _edge(_pack_rows(h1), as1[:, 0], ad1[:, 0], srcp, dstp,
                       _bound_scalar(ms1, md1))
    h2, as2, ad2, ms2, md2 = _merge_dense(
        acc1, den1.reshape(N_PAD, 1), b1.reshape(1, D), W2,
        a2s.reshape(1, D), a2d.reshape(1, D))
    acc2, den2 = _edge(_pack_rows(h2), as2[:, 0], ad2[:, 0], srcp, dstp,
                       _bound_scalar(ms2, md2))
    out = _pool(acc2[:, :N_NODES], den2[:N_NODES].reshape(N_NODES, 1),
                b2.reshape(1, D), bta, btb, Wfc, bfc.reshape(1, 1))
    return out


# speedup vs baseline: 1.3283x; 1.0001x over previous
import jax
import jax.numpy as jnp
from jax import lax
from jax.experimental import pallas as pl
from jax.experimental.pallas import tpu as pltpu
from jax.experimental.pallas import tpu_sc as plsc

N_NODES = 10000
N_PAD = 10240
N_EDGES = 320000
E_PAD = 327680
D = 128
G = 64
DH = 64
NSUB = 16
EPT = E_PAD // NSUB
NCHUNK = EPT // 128
ROWS_PER_TILE = N_PAD // NSUB

_f32 = jnp.float32
_i32 = jnp.int32
_HIMASK = -65536


_R = 1024


def _dense_tail(xv, w_ref, avs_ref, avd_ref,
                h_ref, as_ref, ad_ref, ms_ref, md_ref):
    i = pl.program_id(0)
    h = jnp.dot(xv, w_ref[...], preferred_element_type=_f32)
    h_ref[0] = h[:, :DH]
    h_ref[1] = h[:, DH:]
    asv = jnp.sum(h * avs_ref[...], axis=1, keepdims=True)
    adv = jnp.sum(h * avd_ref[...], axis=1, keepdims=True)
    as_ref[...] = asv
    ad_ref[...] = adv
    bs = jnp.reshape(jnp.max(asv), (1, 1))
    bd = jnp.reshape(jnp.max(adv), (1, 1))

    @pl.when(i == 0)
    def _():
        ms_ref[...] = bs
        md_ref[...] = bd

    @pl.when(i > 0)
    def _():
        ms_ref[...] = jnp.maximum(ms_ref[...], bs)
        md_ref[...] = jnp.maximum(md_ref[...], bd)


_DENSE_OUT_SPECS = [
    pl.BlockSpec((2, _R, DH), lambda i: (0, i, 0)),
    pl.BlockSpec((_R, 1), lambda i: (i, 0)),
    pl.BlockSpec((_R, 1), lambda i: (i, 0)),
    pl.BlockSpec((1, 1), lambda i: (0, 0)),
    pl.BlockSpec((1, 1), lambda i: (0, 0)),
]
_DENSE_OUT_SHAPE = [
    jax.ShapeDtypeStruct((2, N_PAD, DH), _f32),
    jax.ShapeDtypeStruct((N_PAD, 1), _f32),
    jax.ShapeDtypeStruct((N_PAD, 1), _f32),
    jax.ShapeDtypeStruct((1, 1), _f32),
    jax.ShapeDtypeStruct((1, 1), _f32),
]


def _dense_first_body(x_ref, w_ref, avs_ref, avd_ref,
                      h_ref, as_ref, ad_ref, ms_ref, md_ref):
    xv = jnp.maximum(x_ref[...], 0.0)
    _dense_tail(xv, w_ref, avs_ref, avd_ref,
                h_ref, as_ref, ad_ref, ms_ref, md_ref)


def _dense_first(x, W, avs, avd):
    grid = (N_PAD // _R,)
    return pl.pallas_call(
        _dense_first_body,
        grid=grid,
        in_specs=[
            pl.BlockSpec((_R, D), lambda i: (i, 0)),
            pl.BlockSpec((D, D), lambda i: (0, 0)),
            pl.BlockSpec((1, D), lambda i: (0, 0)),
            pl.BlockSpec((1, D), lambda i: (0, 0)),
        ],
        out_specs=_DENSE_OUT_SPECS,
        out_shape=_DENSE_OUT_SHAPE,
    )(x, W, avs, avd)


def _merge_dense_body(acc_ref, den_ref, b_ref, w_ref, avs_ref, avd_ref,
                      h_ref, as_ref, ad_ref, ms_ref, md_ref):
    acc = jnp.concatenate([acc_ref[0], acc_ref[1]], axis=1)
    den = den_ref[...]
    x2 = jnp.maximum(acc / (den + 1e-16) + b_ref[...], 0.0)
    _dense_tail(x2, w_ref, avs_ref, avd_ref,
                h_ref, as_ref, ad_ref, ms_ref, md_ref)


def _merge_dense(acc, den, b, W, avs, avd):
    grid = (N_PAD // _R,)
    return pl.pallas_call(
        _merge_dense_body,
        grid=grid,
        in_specs=[
            pl.BlockSpec((2, _R, DH), lambda i: (0, i, 0)),
            pl.BlockSpec((_R, 1), lambda i: (i, 0)),
            pl.BlockSpec((1, D), lambda i: (0, 0)),
            pl.BlockSpec((D, D), lambda i: (0, 0)),
            pl.BlockSpec((1, D), lambda i: (0, 0)),
            pl.BlockSpec((1, D), lambda i: (0, 0)),
        ],
        out_specs=_DENSE_OUT_SPECS,
        out_shape=_DENSE_OUT_SHAPE,
    )(acc, den, b, W, avs, avd)


_RP = 1000


def _pool_body(acc_ref, den_ref, b_ref, bta_ref, btb_ref, wfc_ref, bfc_ref,
               out_ref, gmx, gsm, gct):
    i = pl.program_id(0)
    nblk = pl.num_programs(0)

    @pl.when(i == 0)
    def _():
        gmx[...] = jnp.zeros((G, D), _f32)
        gsm[...] = jnp.zeros((G, D), _f32)
        gct[...] = jnp.zeros((G, D), _f32)

    acc = jnp.concatenate([acc_ref[0], acc_ref[1]], axis=1)
    den = den_ref[...]
    x3 = jnp.maximum(acc / (den + 1e-16) + b_ref[...], 0.0)
    bb = bta_ref[0]
    btr = btb_ref[0]
    ioc = lax.broadcasted_iota(_i32, (G, _RP), 0)
    oh = (btr == ioc).astype(_f32)
    gsm[...] = gsm[...] + jax.lax.dot_general(
        oh, x3, (((1,), (0,)), ((), ())), preferred_element_type=_f32)
    gct[...] = gct[...] + jax.lax.dot_general(
        oh, jnp.ones_like(x3), (((1,), (0,)), ((), ())),
        preferred_element_type=_f32)
    g0 = jnp.min(bb)
    g1 = jnp.max(bb)

    def body(g, carry):
        mask = bb == g
        vals = jnp.where(mask, x3, 0.0)
        bm = jnp.max(vals, axis=0, keepdims=True)
        gmx[pl.ds(g, 1), :] = jnp.maximum(gmx[pl.ds(g, 1), :], bm)
        return carry

    lax.fori_loop(g0, g1 + 1, body, 0)

    @pl.when(i == nblk - 1)
    def _():
        cnt = jnp.maximum(gct[...], 1.0)
        gmean = gsm[...] / cnt
        pooled = jnp.concatenate([gmx[...], gmean], axis=1)
        out_ref[...] = jnp.dot(pooled, wfc_ref[...],
                               preferred_element_type=_f32) + bfc_ref[...]


def _pool(acc, den, b, bta, btb, Wfc, bfc):
    grid = (N_NODES // _RP,)
    return pl.pallas_call(
        _pool_body,
        grid=grid,
        in_specs=[
            pl.BlockSpec((2, _RP, DH), lambda i: (0, i, 0)),
            pl.BlockSpec((_RP, 1), lambda i: (i, 0)),
            pl.BlockSpec((1, D), lambda i: (0, 0)),
            pl.BlockSpec((1, _RP, 1), lambda i: (i, 0, 0)),
            pl.BlockSpec((1, 1, _RP), lambda i: (i, 0, 0)),
            pl.BlockSpec((2 * D, 1), lambda i: (0, 0)),
            pl.BlockSpec((1, 1), lambda i: (0, 0)),
        ],
        out_specs=pl.BlockSpec((G, 1), lambda i: (0, 0)),
        out_shape=jax.ShapeDtypeStruct((G, 1), _f32),
        scratch_shapes=[
            pltpu.VMEM((G, D), _f32),
            pltpu.VMEM((G, D), _f32),
            pltpu.VMEM((G, D), _f32),
        ],
    )(acc, den, b, bta, btb, Wfc, bfc)


def _edge_body(h_hbm, as_hbm, ad_hbm, src_hbm, dst_hbm, bnd_hbm,
               acc_out, den_out,
               as_v, ad_v, srcb, dstb, exb, exb2, exb3, exb4, exb5,
               rp1, rp2, rp3, rp4, rp5,
               rows, rows2, rows3, rows4, rows5, zb, bnd_v,
               acc_sc, den_sc, gs0, gs1, gs2, gs3, gs4,
               rs0, rs1, rs2, rs3, rs4, ds0, ds1, ds2, ds3, ds4):
    c = lax.axis_index("c")
    s = lax.axis_index("s")
    base_e = s * EPT
    row0 = pl.multiple_of(s * ROWS_PER_TILE, ROWS_PER_TILE)

    pltpu.sync_copy(as_hbm, as_v)
    pltpu.sync_copy(ad_hbm, ad_v)
    pltpu.sync_copy(bnd_hbm, bnd_v)
    b16 = bnd_v[...]
    zero16 = jnp.zeros((16,), _f32)
    iota16 = lax.iota(_i32, 16)

    def zr(r, carry):
        for j in range(DH // 16):
            rows[r, pl.ds(j * 16, 16)] = zero16
        return carry

    lax.fori_loop(0, 128, zr, 0)

    def zz(k, carry):
        zb[pl.ds(pl.multiple_of(k * 16, 16), 16)] = zero16
        return carry

    lax.fori_loop(0, ROWS_PER_TILE // 16, zz, 0)

    for k in range(ROWS_PER_TILE // 128):
        pltpu.sync_copy(rows, acc_sc.at[pl.ds(row0 + k * 128, 128)])
    pltpu.sync_copy(zb, den_sc.at[pl.ds(row0, ROWS_PER_TILE)])
    plsc.subcore_barrier()

    rows_bufs = (rows, rows2, rows3, rows4, rows5)
    ex_bufs = (exb, exb2, exb3, exb4, exb5)
    g_sems = (gs0, gs1, gs2, gs3, gs4)
    r_sems = (rs0, rs1, rs2, rs3, rs4)
    d_sems = (ds0, ds1, ds2, ds3, ds4)
    rp_bufs = (rp1, rp2, rp3, rp4, rp5)
    NBUF = 5
    LOOK = 2
    NCH_P = NCHUNK // 8

    def pass_body(p, pcarry):
        pltpu.sync_copy(src_hbm.at[s, pl.ds(p * NCH_P, NCH_P)], srcb)
        pltpu.sync_copy(dst_hbm.at[s, pl.ds(p * NCH_P, NCH_P)], dstb)

        for b in range(LOOK):
            pltpu.async_copy(
                h_hbm.at[c].at[srcb.at[b]], rp_bufs[b], g_sems[b])

        def pair_body(t, carry):
            for b in range(NBUF):
                ch = NBUF * t + b
                rowsb, rpb, exbb = rows_bufs[b], rp_bufs[b], ex_bufs[b]
                gsem, rsem, dsem = g_sems[b], r_sems[b], d_sems[b]
                ob = (b + LOOK) % NBUF
                orows, orp, oexb = rows_bufs[ob], rp_bufs[ob], ex_bufs[ob]
                ogsem, orsem, odsem = g_sems[ob], r_sems[ob], d_sems[ob]

                @plsc.parallel_loop(0, 8, 1, unroll=8)
                def _scores(g):
                    sl = pl.ds(pl.multiple_of(g * 16, 16), 16)
                    src16 = srcb[ch, sl]
                    dst16 = dstb[ch, sl]
                    sg = plsc.load_gather(as_v, [src16])
                    dg = plsc.load_gather(ad_v, [dst16])
                    sv = sg + dg
                    ev = jnp.where(sv > 0.0, sv, 0.2 * sv)
                    ex = jnp.exp(ev - b16)
                    gidx = (base_e + (p * NCH_P + ch) * 128
                            + g * 16 + iota16)
                    ex = jnp.where(gidx < N_EDGES, ex, 0.0)
                    exbb[0, sl] = ex

                @pl.when(ch + LOOK < NCH_P)
                def _():
                    @pl.when(ch >= NBUF - LOOK)
                    def _():
                        pltpu.make_async_copy(
                            orows, acc_sc.at[dstb.at[ch - (NBUF - LOOK)]],
                            orsem).wait()
                        pltpu.make_async_copy(
                            oexb.at[0],
                            den_sc.at[dstb.at[ch - (NBUF - LOOK)]],
                            odsem).wait()
                    pltpu.async_copy(
                        h_hbm.at[c].at[srcb.at[ch + LOOK]], orp, ogsem)

                pltpu.make_async_copy(
                    h_hbm.at[c].at[srcb.at[ch]], rpb, gsem).wait()

                @plsc.parallel_loop(0, 128, 1, unroll=8)
                def _scale(e):
                    spl = plsc.load_gather(
                        exbb,
                        [jnp.zeros((16,), _i32), jnp.full((16,), e, _i32)])
                    for g in range(2):
                        vi = rpb[e, pl.ds(g * 16, 16)]
                        fa = plsc.bitcast(vi << 16, _f32) * spl
                        fb = plsc.bitcast(vi & _HIMASK, _f32) * spl
                        rowsb[e, pl.ds(g * 32, 16)] = fa
                        rowsb[e, pl.ds(g * 32 + 16, 16)] = fb

                pltpu.async_copy(rowsb, acc_sc.at[dstb.at[ch]], rsem,
                                 add=True)
                pltpu.async_copy(exbb.at[0], den_sc.at[dstb.at[ch]], dsem,
                                 add=True)
            return carry

        lax.fori_loop(0, NCH_P // NBUF, pair_body, 0)
        for b in range(NBUF):
            chl = NCH_P - NBUF + b
            bb = chl % NBUF
            pltpu.make_async_copy(
                rows_bufs[bb], acc_sc.at[dstb.at[chl]], r_sems[bb]).wait()
            pltpu.make_async_copy(
                ex_bufs[bb].at[0], den_sc.at[dstb.at[chl]],
                d_sems[bb]).wait()
        return pcarry

    lax.fori_loop(0, 8, pass_body, 0)
    plsc.subcore_barrier()

    pltpu.sync_copy(acc_sc.at[pl.ds(row0, ROWS_PER_TILE)],
                    acc_out.at[c, pl.ds(row0, ROWS_PER_TILE)])

    @pl.when(c == 0)
    def _():
        pltpu.sync_copy(den_sc.at[pl.ds(row0, ROWS_PER_TILE)],
                        den_out.at[pl.ds(row0, ROWS_PER_TILE)])


def _edge(h_packed, asv, adv, srcp, dstp, bnd16):
    mesh = plsc.VectorSubcoreMesh(core_axis_name="c", subcore_axis_name="s")
    kern = pl.kernel(
        _edge_body,
        out_type=[
            jax.ShapeDtypeStruct((2, N_PAD, DH), _f32),
            jax.ShapeDtypeStruct((N_PAD,), _f32),
        ],
        mesh=mesh,
        scratch_types=[
            pltpu.VMEM((N_PAD,), _f32),
            pltpu.VMEM((N_PAD,), _f32),
            pltpu.VMEM((NCHUNK // 8, 128), _i32),
            pltpu.VMEM((NCHUNK // 8, 128), _i32),
        ] + [pltpu.VMEM((1, 128), _f32)] * 5
          + [pltpu.VMEM((128, DH // 2), _i32)] * 5
          + [pltpu.VMEM((128, DH), _f32)] * 5
          + [
            pltpu.VMEM((ROWS_PER_TILE,), _f32),
            pltpu.VMEM((16,), _f32),
            pltpu.VMEM_SHARED((N_PAD, DH), _f32),
            pltpu.VMEM_SHARED((N_PAD,), _f32),
        ] + [pltpu.SemaphoreType.DMA] * 15,
        compiler_params=pltpu.CompilerParams(
            needs_layout_passes=False, use_tc_tiling_on_sc=False),
    )
    return kern(h_packed, asv, adv, srcp, dstp, bnd16)


def _pack_rows(h):
    hg = h.reshape(2, N_PAD, 2, 2, 16)
    hper = jnp.transpose(hg, (0, 1, 2, 4, 3)).astype(jnp.bfloat16)
    return jax.lax.bitcast_convert_type(hper, _i32).reshape(2, N_PAD, DH // 2)


def _bound_scalar(ms, md):
    b = ms[0, 0] + md[0, 0]
    b = jnp.where(b > 0, b, 0.2 * b)
    return jnp.full((16,), b, _f32)


def kernel(x, edge_index, batch, W1, a1s, a1d, b1, W2, a2s, a2d, b2, Wfc, bfc):
    src = edge_index[0].astype(_i32)
    dst = edge_index[1].astype(_i32)
    padlen = E_PAD - N_EDGES
    srcp = jnp.concatenate([src, jnp.zeros((padlen,), _i32)]).reshape(
        NSUB, NCHUNK, 128)
    dstp = jnp.concatenate([dst, jnp.zeros((padlen,), _i32)]).reshape(
        NSUB, NCHUNK, 128)
    xpad = jnp.pad(x, ((0, N_PAD - N_NODES), (0, 0)))
    bt = batch.astype(_i32)
    bta = bt.reshape(N_NODES // _RP, _RP, 1)
    btb = bt.reshape(N_NODES // _RP, 1, _RP)

    h1, as1, ad1, ms1, md1 = _dense_first(
        xpad, W1, a1s.reshape(1, D), a1d.reshape(1, D))
    acc1, den1 = _edge(_pack_rows(h1), as1[:, 0], ad1[:, 0], srcp, dstp,
                       _bound_scalar(ms1, md1))
    h2, as2, ad2, ms2, md2 = _merge_dense(
        acc1, den1.reshape(N_PAD, 1), b1.reshape(1, D), W2,
        a2s.reshape(1, D), a2d.reshape(1, D))
    acc2, den2 = _edge(_pack_rows(h2), as2[:, 0], ad2[:, 0], srcp, dstp,
                       _bound_scalar(ms2, md2))
    out = _pool(acc2[:, :N_NODES], den2[:N_NODES].reshape(N_NODES, 1),
                b2.reshape(1, D), bta, btb, Wfc, bfc.reshape(1, 1))
    return out
